# baseline (speedup 1.0000x reference)
import functools

import jax
import jax.numpy as jnp
from jax import lax
from jax.experimental import pallas as pl
from jax.experimental.pallas import tpu as pltpu
from jax.experimental.pallas import tpu_sc as plsc

NC = 2
NS = 16
NW = NC * NS
L = 16

B = 1024
R = 160
D = 128
BPW = B // NW
SLICE = 80
SPB = R // SLICE
NSL = BPW * SPB
DV = D // L


def _wid():
    return lax.axis_index("s") * NC + lax.axis_index("c")


def _sum_body(tok_hbm, table_hbm, sums_hbm, idx_v, rows_v, sums_v, sem):
    w = _wid()
    pltpu.sync_copy(tok_hbm.at[w], idx_v)

    @pl.loop(0, BPW)
    def _per_b(bl):
        for h in range(SPB):
            pltpu.async_copy(
                table_hbm.at[idx_v.at[bl * SPB + h]],
                rows_v.at[pl.ds(h * SLICE, SLICE)],
                sem,
            ).wait()

        def acc_row(r, carry):
            return tuple(carry[j] + rows_v[r, pl.ds(j * L, L)]
                         for j in range(DV))

        zeros = tuple(jnp.zeros((L,), jnp.float32) for _ in range(DV))
        acc = lax.fori_loop(0, R, acc_row, zeros)
        for j in range(DV):
            sums_v[bl, pl.ds(j * L, L)] = acc[j]

    pltpu.sync_copy(sums_v, sums_hbm.at[pl.ds(w * BPW, BPW)])


def _add_body(tok_hbm, table_hbm, te_hbm, out_hbm, idx_v, te_v, rows_v, sem):
    w = _wid()
    pltpu.sync_copy(tok_hbm.at[w], idx_v)
    pltpu.sync_copy(te_hbm.at[pl.ds(w * BPW, BPW)], te_v)

    @pl.loop(0, BPW)
    def _per_b(bl):
        for h in range(SPB):
            pltpu.async_copy(
                table_hbm.at[idx_v.at[bl * SPB + h]],
                rows_v.at[pl.ds(h * SLICE, SLICE)],
                sem,
            ).wait()
        te = tuple(te_v[bl, pl.ds(j * L, L)] for j in range(DV))

        @pl.loop(0, R)
        def _per_row(r):
            for j in range(DV):
                rows_v[r, pl.ds(j * L, L)] = rows_v[r, pl.ds(j * L, L)] + te[j]

        pltpu.sync_copy(rows_v,
                        out_hbm.at[pl.ds((w * BPW + bl) * R, R)])


def _mlp_body(sums_ref, gid_ref, known_ref, w1_ref, b1_ref, w2_ref, b2_ref,
              br_ref, te_ref):
    mean = sums_ref[...] * (1.0 / R)
    h = jnp.dot(mean, w1_ref[...].T, preferred_element_type=jnp.float32)
    h = jnp.maximum(h + b1_ref[...], 0.0)
    infer = jnp.dot(h, w2_ref[...].T, preferred_element_type=jnp.float32)
    infer = infer + b2_ref[...]
    gid = gid_ref[...]
    onehot = (gid == lax.broadcasted_iota(jnp.int32, (B, 64), 1))
    known = jnp.dot(onehot.astype(jnp.float32), known_ref[...],
                    preferred_element_type=jnp.float32)
    ratio = br_ref[0]
    te_ref[...] = (known * (1.0 - ratio) + infer) * ratio


def kernel(obs_tokens, game_ids, token_table, known_table, W1, b1, W2, b2,
           blend_ratio):
    Bh, Th, Kh = obs_tokens.shape
    tok = obs_tokens.reshape(NW, NSL, SLICE)

    mesh = plsc.VectorSubcoreMesh(core_axis_name="c", subcore_axis_name="s",
                                  num_cores=NC, num_subcores=NS)

    sum_k = pl.kernel(
        _sum_body,
        out_type=jax.ShapeDtypeStruct((B, D), jnp.float32),
        mesh=mesh,
        scratch_types=[
            pltpu.VMEM((NSL, SLICE), jnp.int32),
            pltpu.VMEM((R, D), jnp.float32),
            pltpu.VMEM((BPW, D), jnp.float32),
            pltpu.SemaphoreType.DMA,
        ],
    )
    sums = sum_k(tok, token_table)

    known_pad = jnp.zeros((64, D), jnp.float32).at[:known_table.shape[0]].set(
        known_table)
    te = pl.pallas_call(
        _mlp_body,
        out_shape=jax.ShapeDtypeStruct((B, D), jnp.float32),
        in_specs=[
            pl.BlockSpec(memory_space=pltpu.VMEM),
            pl.BlockSpec(memory_space=pltpu.VMEM),
            pl.BlockSpec(memory_space=pltpu.VMEM),
            pl.BlockSpec(memory_space=pltpu.VMEM),
            pl.BlockSpec(memory_space=pltpu.VMEM),
            pl.BlockSpec(memory_space=pltpu.VMEM),
            pl.BlockSpec(memory_space=pltpu.VMEM),
            pl.BlockSpec(memory_space=pltpu.SMEM),
        ],
        out_specs=pl.BlockSpec(memory_space=pltpu.VMEM),
    )(sums, game_ids.reshape(B, 1), known_pad, W1, b1.reshape(1, D), W2,
      b2.reshape(1, D), blend_ratio.reshape(1))

    add_k = pl.kernel(
        _add_body,
        out_type=jax.ShapeDtypeStruct((B * R, D), jnp.float32),
        mesh=mesh,
        scratch_types=[
            pltpu.VMEM((NSL, SLICE), jnp.int32),
            pltpu.VMEM((BPW, D), jnp.float32),
            pltpu.VMEM((R, D), jnp.float32),
            pltpu.SemaphoreType.DMA,
        ],
    )
    out = add_k(tok, token_table, te)
    return out.reshape(Bh, Th * Kh, D)


# speedup vs baseline: 1.7711x; 1.7711x over previous
import jax
import jax.numpy as jnp
from jax import lax
from jax.experimental import pallas as pl
from jax.experimental.pallas import tpu as pltpu
from jax.experimental.pallas import tpu_sc as plsc

NC = 2
NS = 16
NW = NC * NS
L = 16

B = 1024
R = 160
D = 128
BPW = B // NW
SLICE = 80
SPB = R // SLICE
NSL = BPW * SPB
DV = D // L
NB = 8
DEPTH = 4


def _wid():
    return lax.axis_index("s") * NC + lax.axis_index("c")


def _sum_body(tok_hbm, table_hbm, sums_hbm, idx_v, rows_v, sums_v, gsem):
    w = _wid()
    pltpu.sync_copy(tok_hbm.at[w], idx_v)

    def fire(s):
        return pltpu.async_copy(table_hbm.at[idx_v.at[s]],
                                rows_v.at[s % NB], gsem.at[s % NB])

    gd = [None] * NSL
    for s in range(DEPTH):
        gd[s] = fire(s)

    acc = None
    for s in range(NSL):
        if s + DEPTH < NSL:
            gd[s + DEPTH] = fire(s + DEPTH)
        gd[s].wait()
        buf = rows_v.at[s % NB]
        if s % SPB == 0:
            acc = tuple(jnp.zeros((L,), jnp.float32) for _ in range(DV))

        def acc_row(r, carry, buf=buf):
            return tuple(carry[j] + buf[r, pl.ds(j * L, L)]
                         for j in range(DV))

        acc = lax.fori_loop(0, SLICE, acc_row, acc, unroll=4)
        if s % SPB == SPB - 1:
            for j in range(DV):
                sums_v[s // SPB, pl.ds(j * L, L)] = acc[j]

    pltpu.sync_copy(sums_v, sums_hbm.at[pl.ds(w * BPW, BPW)])


def _add_body(tok_hbm, table_hbm, te_hbm, out_hbm, idx_v, te_v, rows_v,
              gsem, ssem):
    w = _wid()
    pltpu.sync_copy(tok_hbm.at[w], idx_v)
    pltpu.sync_copy(te_hbm.at[pl.ds(w * BPW, BPW)], te_v)

    def fire(s):
        return pltpu.async_copy(table_hbm.at[idx_v.at[s]],
                                rows_v.at[s % NB], gsem.at[s % NB])

    gd = [None] * NSL
    sd = [None] * NSL
    for s in range(DEPTH):
        gd[s] = fire(s)

    for s in range(NSL):
        if s + DEPTH < NSL:
            if s + DEPTH >= NB and sd[s + DEPTH - NB] is not None:
                sd[s + DEPTH - NB].wait()
            gd[s + DEPTH] = fire(s + DEPTH)
        gd[s].wait()
        buf = rows_v.at[s % NB]
        te = tuple(te_v[s // SPB, pl.ds(j * L, L)] for j in range(DV))

        def add_row(r, carry, buf=buf, te=te):
            for j in range(DV):
                buf[r, pl.ds(j * L, L)] = buf[r, pl.ds(j * L, L)] + te[j]
            return carry

        lax.fori_loop(0, SLICE, add_row, 0, unroll=4)
        sd[s] = pltpu.async_copy(
            rows_v.at[s % NB],
            out_hbm.at[pl.ds((w * NSL + s) * SLICE, SLICE)],
            ssem.at[s % NB])

    for s in range(NSL - NB, NSL):
        if sd[s] is not None:
            sd[s].wait()


def _mlp_body(sums_ref, gid_ref, known_ref, w1_ref, b1_ref, w2_ref, b2_ref,
              br_ref, te_ref):
    mean = sums_ref[...] * (1.0 / R)
    h = jnp.dot(mean, w1_ref[...].T, preferred_element_type=jnp.float32)
    h = jnp.maximum(h + b1_ref[...], 0.0)
    infer = jnp.dot(h, w2_ref[...].T, preferred_element_type=jnp.float32)
    infer = infer + b2_ref[...]
    gid = gid_ref[...]
    onehot = (gid == lax.broadcasted_iota(jnp.int32, (B, 64), 1))
    known = jnp.dot(onehot.astype(jnp.float32), known_ref[...],
                    preferred_element_type=jnp.float32)
    ratio = br_ref[0]
    te_ref[...] = (known * (1.0 - ratio) + infer) * ratio


def kernel(obs_tokens, game_ids, token_table, known_table, W1, b1, W2, b2,
           blend_ratio):
    Bh, Th, Kh = obs_tokens.shape
    tok = obs_tokens.reshape(NW, NSL, SLICE)

    mesh = plsc.VectorSubcoreMesh(core_axis_name="c", subcore_axis_name="s",
                                  num_cores=NC, num_subcores=NS)

    sum_k = pl.kernel(
        _sum_body,
        out_type=jax.ShapeDtypeStruct((B, D), jnp.float32),
        mesh=mesh,
        scratch_types=[
            pltpu.VMEM((NSL, SLICE), jnp.int32),
            pltpu.VMEM((NB, SLICE, D), jnp.float32),
            pltpu.VMEM((BPW, D), jnp.float32),
            pltpu.SemaphoreType.DMA((NB,)),
        ],
    )
    sums = sum_k(tok, token_table)

    known_pad = jnp.zeros((64, D), jnp.float32).at[:known_table.shape[0]].set(
        known_table)
    te = pl.pallas_call(
        _mlp_body,
        out_shape=jax.ShapeDtypeStruct((B, D), jnp.float32),
        in_specs=[
            pl.BlockSpec(memory_space=pltpu.VMEM),
            pl.BlockSpec(memory_space=pltpu.VMEM),
            pl.BlockSpec(memory_space=pltpu.VMEM),
            pl.BlockSpec(memory_space=pltpu.VMEM),
            pl.BlockSpec(memory_space=pltpu.VMEM),
            pl.BlockSpec(memory_space=pltpu.VMEM),
            pl.BlockSpec(memory_space=pltpu.VMEM),
            pl.BlockSpec(memory_space=pltpu.SMEM),
        ],
        out_specs=pl.BlockSpec(memory_space=pltpu.VMEM),
    )(sums, game_ids.reshape(B, 1), known_pad, W1, b1.reshape(1, D), W2,
      b2.reshape(1, D), blend_ratio.reshape(1))

    add_k = pl.kernel(
        _add_body,
        out_type=jax.ShapeDtypeStruct((B * R, D), jnp.float32),
        mesh=mesh,
        scratch_types=[
            pltpu.VMEM((NSL, SLICE), jnp.int32),
            pltpu.VMEM((BPW, D), jnp.float32),
            pltpu.VMEM((NB, SLICE, D), jnp.float32),
            pltpu.SemaphoreType.DMA((NB,)),
            pltpu.SemaphoreType.DMA((NB,)),
        ],
    )
    out = add_k(tok, token_table, te)
    return out.reshape(Bh, Th * Kh, D)
